# baseline (speedup 1.0000x reference)
import functools

import jax
import jax.numpy as jnp
from jax import lax
from jax.experimental import pallas as pl
from jax.experimental.pallas import tpu as pltpu
from jax.experimental.pallas import tpu_sc as plsc

N_FEAT = 256
NWORD = N_FEAT // 2
L = 16
NC = 2
NS = 16
NW = NC * NS
C = 128


def _dot_kernel(src_hbm, dst_hbm, ufeat_hbm, ifeat_hbm, out_hbm,
                sidx, didx, u0, u1, v0, v1, o0, o1,
                su0, su1, sv0, sv1, so0, so1):
    wid = lax.axis_index("s") * NC + lax.axis_index("c")
    e_per_w = src_hbm.shape[0] // NW
    n_chunks = e_per_w // C
    w_base = wid * e_per_w
    row_ids = lax.iota(jnp.int32, L)
    ubuf = (u0, u1)
    vbuf = (v0, v1)
    obuf = (o0, o1)
    usem = (su0, su1)
    vsem = (sv0, sv1)
    osem = (so0, so1)

    pltpu.sync_copy(src_hbm.at[pl.ds(w_base, e_per_w)], sidx)
    pltpu.sync_copy(dst_hbm.at[pl.ds(w_base, e_per_w)], didx)

    def fire(i, b):
        off = i * C
        pltpu.async_copy(ufeat_hbm.at[sidx.at[pl.ds(off, C)]], ubuf[b], usem[b])
        pltpu.async_copy(ifeat_hbm.at[didx.at[pl.ds(off, C)]], vbuf[b], vsem[b])

    def wait_rows(b):
        pltpu.make_async_copy(
            ufeat_hbm.at[sidx.at[pl.ds(0, C)]], ubuf[b], usem[b]).wait()
        pltpu.make_async_copy(
            ifeat_hbm.at[didx.at[pl.ds(0, C)]], vbuf[b], vsem[b]).wait()

    def wait_out(b):
        pltpu.make_async_copy(
            obuf[b], out_hbm.at[pl.ds(0, C)], osem[b]).wait()

    fire(0, 0)

    def outer_body(o, _):
        for b in range(2):
            i = o * 2 + b
            wait_rows(b)

            @pl.when(i + 1 < n_chunks)
            def _():
                fire(i + 1, 1 - b)

            @pl.when(i >= 2)
            def _():
                wait_out(b)

            urows = ubuf[b]
            vrows = vbuf[b]

            def group_body(g, _):
                base = g * L
                res = jnp.zeros((L,), jnp.float32)
                hi_mask = jnp.full((L,), -65536, jnp.int32)
                for e in range(L):
                    acc = jnp.zeros((L,), jnp.float32)
                    for j in range(NWORD // L):
                        uw = urows[base + e, pl.ds(j * L, L)]
                        vw = vrows[base + e, pl.ds(j * L, L)]
                        uhi = plsc.bitcast(uw & hi_mask, jnp.float32)
                        ulo = plsc.bitcast(uw << 16, jnp.float32)
                        vhi = plsc.bitcast(vw & hi_mask, jnp.float32)
                        vlo = plsc.bitcast(vw << 16, jnp.float32)
                        acc += uhi * vhi
                        acc += ulo * vlo
                    s = jnp.sum(acc)
                    res = jnp.where(row_ids == e, s, res)
                obuf[b][pl.ds(base, L)] = res
                return 0

            lax.fori_loop(0, C // L, group_body, 0)
            pltpu.async_copy(obuf[b], out_hbm.at[pl.ds(w_base + i * C, C)],
                             osem[b])
        return 0

    lax.fori_loop(0, n_chunks // 2, outer_body, 0)
    wait_out(0)
    wait_out(1)


def kernel(ufeat, ifeat, Q, edge_index):
    del Q
    e = edge_index.shape[1]
    src = edge_index[0].astype(jnp.int32)
    dst = edge_index[1].astype(jnp.int32)
    n_u, n_i = ufeat.shape[0], ifeat.shape[0]
    ufeat = lax.bitcast_convert_type(
        ufeat.astype(jnp.bfloat16).reshape(n_u, NWORD, 2), jnp.int32)
    ifeat = lax.bitcast_convert_type(
        ifeat.astype(jnp.bfloat16).reshape(n_i, NWORD, 2), jnp.int32)
    blk = NW * C * 2
    e_pad = ((e + blk - 1) // blk) * blk
    if e_pad != e:
        src = jnp.pad(src, (0, e_pad - e))
        dst = jnp.pad(dst, (0, e_pad - e))
    e_per_w = e_pad // NW

    run = pl.kernel(
        _dot_kernel,
        out_type=jax.ShapeDtypeStruct((e_pad,), jnp.float32),
        mesh=plsc.VectorSubcoreMesh(
            core_axis_name="c", subcore_axis_name="s",
            num_cores=NC, num_subcores=NS),
        scratch_types=[
            pltpu.VMEM((e_per_w,), jnp.int32),
            pltpu.VMEM((e_per_w,), jnp.int32),
            pltpu.VMEM((C, NWORD), jnp.int32),
            pltpu.VMEM((C, NWORD), jnp.int32),
            pltpu.VMEM((C, NWORD), jnp.int32),
            pltpu.VMEM((C, NWORD), jnp.int32),
            pltpu.VMEM((C,), jnp.float32),
            pltpu.VMEM((C,), jnp.float32),
            pltpu.SemaphoreType.DMA,
            pltpu.SemaphoreType.DMA,
            pltpu.SemaphoreType.DMA,
            pltpu.SemaphoreType.DMA,
            pltpu.SemaphoreType.DMA,
            pltpu.SemaphoreType.DMA,
        ],
        compiler_params=pltpu.CompilerParams(needs_layout_passes=False),
    )
    out = run(src, dst, ufeat, ifeat)
    return out[:e, None]


# speedup vs baseline: 1.0203x; 1.0203x over previous
import functools

import jax
import jax.numpy as jnp
from jax import lax
from jax.experimental import pallas as pl
from jax.experimental.pallas import tpu as pltpu
from jax.experimental.pallas import tpu_sc as plsc

N_FEAT = 256
L = 16
NC = 2
NS = 16
C = 64
N0 = 116
N1 = 42
E_PAD = NS * (N0 + N1) * C


def _dot_kernel(src_hbm, dst_hbm, ufeat_hbm, ifeat_hbm, out_hbm,
                sidx, didx, u0, u1, v0, v1, o0, o1,
                su0, su1, sv0, sv1, so0, so1):
    cid = lax.axis_index("c")
    sid = lax.axis_index("s")
    row_ids = lax.iota(jnp.int32, L)
    ubuf = (u0, u1)
    vbuf = (v0, v1)
    obuf = (o0, o1)
    usem = (su0, su1)
    vsem = (sv0, sv1)
    osem = (so0, so1)

    def run_core(n_chunks, w_base):
        e_per_w = n_chunks * C
        pltpu.sync_copy(src_hbm.at[pl.ds(w_base, e_per_w)],
                        sidx.at[pl.ds(0, e_per_w)])
        pltpu.sync_copy(dst_hbm.at[pl.ds(w_base, e_per_w)],
                        didx.at[pl.ds(0, e_per_w)])

        def fire(i, b):
            off = i * C
            pltpu.async_copy(
                ufeat_hbm.at[sidx.at[pl.ds(off, C)]], ubuf[b], usem[b])
            pltpu.async_copy(
                ifeat_hbm.at[didx.at[pl.ds(off, C)]], vbuf[b], vsem[b])

        def wait_rows(b):
            pltpu.make_async_copy(
                ufeat_hbm.at[sidx.at[pl.ds(0, C)]], ubuf[b], usem[b]).wait()
            pltpu.make_async_copy(
                ifeat_hbm.at[didx.at[pl.ds(0, C)]], vbuf[b], vsem[b]).wait()

        def wait_out(b):
            pltpu.make_async_copy(
                obuf[b], out_hbm.at[pl.ds(0, C)], osem[b]).wait()

        fire(0, 0)

        def outer_body(o, _):
            for b in range(2):
                i = o * 2 + b
                wait_rows(b)

                @pl.when(i + 1 < n_chunks)
                def _():
                    fire(i + 1, 1 - b)

                @pl.when(i >= 2)
                def _():
                    wait_out(b)

                urows = ubuf[b]
                vrows = vbuf[b]

                def group_body(g, _):
                    base = g * L
                    res = jnp.zeros((L,), jnp.float32)
                    for e in range(L):
                        acc = (urows[base + e, pl.ds(0, L)]
                               * vrows[base + e, pl.ds(0, L)])
                        for j in range(1, N_FEAT // L):
                            acc += (urows[base + e, pl.ds(j * L, L)]
                                    * vrows[base + e, pl.ds(j * L, L)])
                        s = jnp.sum(acc)
                        res = jnp.where(row_ids == e, s, res)
                    obuf[b][pl.ds(base, L)] = res
                    return 0

                lax.fori_loop(0, C // L, group_body, 0)
                pltpu.async_copy(
                    obuf[b], out_hbm.at[pl.ds(w_base + i * C, C)], osem[b])
            return 0

        lax.fori_loop(0, n_chunks // 2, outer_body, 0)
        wait_out(0)
        wait_out(1)

    @pl.when(cid == 0)
    def _():
        run_core(N0, sid * (N0 * C))

    @pl.when(cid == 1)
    def _():
        run_core(N1, NS * (N0 * C) + sid * (N1 * C))


def kernel(ufeat, ifeat, Q, edge_index):
    del Q
    e = edge_index.shape[1]
    assert e <= E_PAD
    src = edge_index[0].astype(jnp.int32)
    dst = edge_index[1].astype(jnp.int32)
    if E_PAD != e:
        src = jnp.pad(src, (0, E_PAD - e))
        dst = jnp.pad(dst, (0, E_PAD - e))

    run = pl.kernel(
        _dot_kernel,
        out_type=jax.ShapeDtypeStruct((E_PAD,), jnp.float32),
        mesh=plsc.VectorSubcoreMesh(
            core_axis_name="c", subcore_axis_name="s",
            num_cores=NC, num_subcores=NS),
        scratch_types=[
            pltpu.VMEM((N0 * C,), jnp.int32),
            pltpu.VMEM((N0 * C,), jnp.int32),
            pltpu.VMEM((C, N_FEAT), jnp.float32),
            pltpu.VMEM((C, N_FEAT), jnp.float32),
            pltpu.VMEM((C, N_FEAT), jnp.float32),
            pltpu.VMEM((C, N_FEAT), jnp.float32),
            pltpu.VMEM((C,), jnp.float32),
            pltpu.VMEM((C,), jnp.float32),
            pltpu.SemaphoreType.DMA,
            pltpu.SemaphoreType.DMA,
            pltpu.SemaphoreType.DMA,
            pltpu.SemaphoreType.DMA,
            pltpu.SemaphoreType.DMA,
            pltpu.SemaphoreType.DMA,
        ],
        compiler_params=pltpu.CompilerParams(needs_layout_passes=False),
    )
    out = run(src, dst, ufeat, ifeat)
    return out[:e, None]


# speedup vs baseline: 2.6592x; 2.6063x over previous
import functools

import jax
import jax.numpy as jnp
from jax import lax
from jax.experimental import pallas as pl
from jax.experimental.pallas import tpu as pltpu
from jax.experimental.pallas import tpu_sc as plsc

N_FEAT = 256
NWORD = N_FEAT // 2
L = 16
NC = 2
NS = 16
C = 128
N0 = 58
N1 = 21
E_PAD = NS * (N0 + N1) * C


def _dot_kernel(src_hbm, dst_hbm, ufeat_hbm, ifeat_hbm, out_hbm,
                sidx, didx, u0, u1, v0, v1, o0, o1,
                su0, su1, sv0, sv1, so0, so1):
    cid = lax.axis_index("c")
    sid = lax.axis_index("s")
    row_ids = lax.iota(jnp.int32, L)
    hi_mask = jnp.full((L,), -65536, jnp.int32)
    ubuf = (u0, u1)
    vbuf = (v0, v1)
    obuf = (o0, o1)
    usem = (su0, su1)
    vsem = (sv0, sv1)
    osem = (so0, so1)

    def run_core(n_chunks, w_base):
        e_per_w = n_chunks * C
        pltpu.sync_copy(src_hbm.at[pl.ds(w_base, e_per_w)],
                        sidx.at[pl.ds(0, e_per_w)])
        pltpu.sync_copy(dst_hbm.at[pl.ds(w_base, e_per_w)],
                        didx.at[pl.ds(0, e_per_w)])

        def fire(i, b):
            off = i * C
            pltpu.async_copy(
                ufeat_hbm.at[sidx.at[pl.ds(off, C)]], ubuf[b], usem[b])
            pltpu.async_copy(
                ifeat_hbm.at[didx.at[pl.ds(off, C)]], vbuf[b], vsem[b])

        def wait_rows(b):
            pltpu.make_async_copy(
                ufeat_hbm.at[sidx.at[pl.ds(0, C)]], ubuf[b], usem[b]).wait()
            pltpu.make_async_copy(
                ifeat_hbm.at[didx.at[pl.ds(0, C)]], vbuf[b], vsem[b]).wait()

        def wait_out(b):
            pltpu.make_async_copy(
                obuf[b], out_hbm.at[pl.ds(0, C)], osem[b]).wait()

        def compute(i, b, guarded):
            wait_rows(b)
            if guarded:
                @pl.when(i + 1 < n_chunks)
                def _():
                    fire(i + 1, 1 - b)

            @pl.when(i >= 2)
            def _():
                wait_out(b)

            urows = ubuf[b]
            vrows = vbuf[b]

            def group_body(g, _):
                base = g * L
                res = jnp.zeros((L,), jnp.float32)
                for e in range(L):
                    acc = jnp.zeros((L,), jnp.float32)
                    for j in range(NWORD // L):
                        uw = urows[base + e, pl.ds(j * L, L)]
                        vw = vrows[base + e, pl.ds(j * L, L)]
                        uhi = plsc.bitcast(uw & hi_mask, jnp.float32)
                        ulo = plsc.bitcast(uw << 16, jnp.float32)
                        vhi = plsc.bitcast(vw & hi_mask, jnp.float32)
                        vlo = plsc.bitcast(vw << 16, jnp.float32)
                        acc += uhi * vhi
                        acc += ulo * vlo
                    s = jnp.sum(acc)
                    res = jnp.where(row_ids == e, s, res)
                obuf[b][pl.ds(base, L)] = res
                return 0

            lax.fori_loop(0, C // L, group_body, 0)
            pltpu.async_copy(
                obuf[b], out_hbm.at[pl.ds(w_base + i * C, C)], osem[b])

        fire(0, 0)

        def outer_body(o, _):
            for b in range(2):
                compute(o * 2 + b, b, guarded=True)
            return 0

        lax.fori_loop(0, n_chunks // 2, outer_body, 0)
        if n_chunks % 2:
            compute(n_chunks - 1, 0, guarded=False)
        wait_out(0)
        wait_out(1)

    @pl.when(cid == 0)
    def _():
        run_core(N0, sid * (N0 * C))

    @pl.when(cid == 1)
    def _():
        run_core(N1, NS * (N0 * C) + sid * (N1 * C))


def kernel(ufeat, ifeat, Q, edge_index):
    del Q
    e = edge_index.shape[1]
    assert e <= E_PAD
    src = edge_index[0].astype(jnp.int32)
    dst = edge_index[1].astype(jnp.int32)
    if E_PAD != e:
        src = jnp.pad(src, (0, E_PAD - e))
        dst = jnp.pad(dst, (0, E_PAD - e))

    def pack(x):
        xb = x.astype(jnp.bfloat16)
        hi = lax.bitcast_convert_type(xb[:, :NWORD], jnp.uint16)
        lo = lax.bitcast_convert_type(xb[:, NWORD:], jnp.uint16)
        packed = (hi.astype(jnp.uint32) << 16) | lo.astype(jnp.uint32)
        return lax.bitcast_convert_type(packed, jnp.int32)

    run = pl.kernel(
        _dot_kernel,
        out_type=jax.ShapeDtypeStruct((E_PAD,), jnp.float32),
        mesh=plsc.VectorSubcoreMesh(
            core_axis_name="c", subcore_axis_name="s",
            num_cores=NC, num_subcores=NS),
        scratch_types=[
            pltpu.VMEM((N0 * C,), jnp.int32),
            pltpu.VMEM((N0 * C,), jnp.int32),
            pltpu.VMEM((C, NWORD), jnp.int32),
            pltpu.VMEM((C, NWORD), jnp.int32),
            pltpu.VMEM((C, NWORD), jnp.int32),
            pltpu.VMEM((C, NWORD), jnp.int32),
            pltpu.VMEM((C,), jnp.float32),
            pltpu.VMEM((C,), jnp.float32),
            pltpu.SemaphoreType.DMA,
            pltpu.SemaphoreType.DMA,
            pltpu.SemaphoreType.DMA,
            pltpu.SemaphoreType.DMA,
            pltpu.SemaphoreType.DMA,
            pltpu.SemaphoreType.DMA,
        ],
        compiler_params=pltpu.CompilerParams(needs_layout_passes=False),
    )
    out = run(src, dst, pack(ufeat), pack(ifeat))
    return out[:e, None]


# speedup vs baseline: 2.8294x; 1.0640x over previous
import jax
import jax.numpy as jnp
from jax import lax
from jax.experimental import pallas as pl
from jax.experimental.pallas import tpu as pltpu
from jax.experimental.pallas import tpu_sc as plsc

N_FEAT = 256
NWORD = N_FEAT // 2
L = 16
NC = 2
NS = 16
C = 80
N0 = 94
N1 = 31
E_PAD = NS * (N0 + N1) * C


def _dot_kernel(src_hbm, dst_hbm, ufeat_hbm, ifeat_hbm, out_hbm,
                sidx, didx, u0, u1, v0, v1, o0, o1,
                su0, su1, sv0, sv1, so0, so1):
    cid = lax.axis_index("c")
    sid = lax.axis_index("s")
    row_ids = lax.iota(jnp.int32, L)
    ubuf = (u0, u1)
    vbuf = (v0, v1)
    obuf = (o0, o1)
    usem = (su0, su1)
    vsem = (sv0, sv1)
    osem = (so0, so1)

    def run_core(n_chunks, w_base):
        e_per_w = n_chunks * C
        pltpu.sync_copy(src_hbm.at[pl.ds(w_base, e_per_w)],
                        sidx.at[pl.ds(0, e_per_w)])
        pltpu.sync_copy(dst_hbm.at[pl.ds(w_base, e_per_w)],
                        didx.at[pl.ds(0, e_per_w)])

        def fire(i, b):
            off = i * C
            pltpu.async_copy(
                ufeat_hbm.at[sidx.at[pl.ds(off, C)]], ubuf[b], usem[b])
            pltpu.async_copy(
                ifeat_hbm.at[didx.at[pl.ds(off, C)]], vbuf[b], vsem[b])

        def wait_rows(b):
            pltpu.make_async_copy(
                ufeat_hbm.at[sidx.at[pl.ds(0, C)]], ubuf[b], usem[b]).wait()
            pltpu.make_async_copy(
                ifeat_hbm.at[didx.at[pl.ds(0, C)]], vbuf[b], vsem[b]).wait()

        def wait_out(b):
            pltpu.make_async_copy(
                obuf[b], out_hbm.at[pl.ds(0, C)], osem[b]).wait()

        def compute(i, b, guarded):
            wait_rows(b)
            if guarded:
                @pl.when(i + 1 < n_chunks)
                def _():
                    fire(i + 1, 1 - b)

            @pl.when(i >= 2)
            def _():
                wait_out(b)

            urows = ubuf[b]
            vrows = vbuf[b]

            def group_body(g, _):
                base = g * L
                res = jnp.zeros((L,), jnp.float32)
                for e in range(L):
                    acc = jnp.zeros((L,), jnp.float32)
                    for j in range(NWORD // L):
                        uw = urows[base + e, pl.ds(j * L, L)]
                        vw = vrows[base + e, pl.ds(j * L, L)]
                        uhi = plsc.bitcast(uw, jnp.float32)
                        ulo = plsc.bitcast(uw << 16, jnp.float32)
                        vhi = plsc.bitcast(vw, jnp.float32)
                        vlo = plsc.bitcast(vw << 16, jnp.float32)
                        acc += uhi * vhi
                        acc += ulo * vlo
                    s = jnp.sum(acc)
                    res = jnp.where(row_ids == e, s, res)
                obuf[b][pl.ds(base, L)] = res
                return 0

            lax.fori_loop(0, C // L, group_body, 0)
            pltpu.async_copy(
                obuf[b], out_hbm.at[pl.ds(w_base + i * C, C)], osem[b])

        fire(0, 0)

        def outer_body(o, _):
            for b in range(2):
                compute(o * 2 + b, b, guarded=True)
            return 0

        lax.fori_loop(0, n_chunks // 2, outer_body, 0)
        if n_chunks % 2:
            compute(n_chunks - 1, 0, guarded=False)
        wait_out(0)
        wait_out(1)

    @pl.when(cid == 0)
    def _():
        run_core(N0, sid * (N0 * C))

    @pl.when(cid == 1)
    def _():
        run_core(N1, NS * (N0 * C) + sid * (N1 * C))


def kernel(ufeat, ifeat, Q, edge_index):
    del Q
    e = edge_index.shape[1]
    assert e <= E_PAD
    src = edge_index[0].astype(jnp.int32)
    dst = edge_index[1].astype(jnp.int32)
    if E_PAD != e:
        src = jnp.pad(src, (0, E_PAD - e))
        dst = jnp.pad(dst, (0, E_PAD - e))

    def pack(x):
        xb = x.astype(jnp.bfloat16)
        hi = lax.bitcast_convert_type(xb[:, :NWORD], jnp.uint16)
        lo = lax.bitcast_convert_type(xb[:, NWORD:], jnp.uint16)
        packed = (hi.astype(jnp.uint32) << 16) | lo.astype(jnp.uint32)
        return lax.bitcast_convert_type(packed, jnp.int32)

    run = pl.kernel(
        _dot_kernel,
        out_type=jax.ShapeDtypeStruct((E_PAD,), jnp.float32),
        mesh=plsc.VectorSubcoreMesh(
            core_axis_name="c", subcore_axis_name="s",
            num_cores=NC, num_subcores=NS),
        scratch_types=[
            pltpu.VMEM((N0 * C,), jnp.int32),
            pltpu.VMEM((N0 * C,), jnp.int32),
            pltpu.VMEM((C, NWORD), jnp.int32),
            pltpu.VMEM((C, NWORD), jnp.int32),
            pltpu.VMEM((C, NWORD), jnp.int32),
            pltpu.VMEM((C, NWORD), jnp.int32),
            pltpu.VMEM((C,), jnp.float32),
            pltpu.VMEM((C,), jnp.float32),
            pltpu.SemaphoreType.DMA,
            pltpu.SemaphoreType.DMA,
            pltpu.SemaphoreType.DMA,
            pltpu.SemaphoreType.DMA,
            pltpu.SemaphoreType.DMA,
            pltpu.SemaphoreType.DMA,
        ],
        compiler_params=pltpu.CompilerParams(needs_layout_passes=False),
    )
    out = run(src, dst, pack(ufeat), pack(ifeat))
    return out[:e, None]


# speedup vs baseline: 3.7849x; 1.3377x over previous
import jax
import jax.numpy as jnp
from jax import lax
from jax.experimental import pallas as pl
from jax.experimental.pallas import tpu as pltpu
from jax.experimental.pallas import tpu_sc as plsc

N_FEAT = 256
NWORD = N_FEAT // 2
L = 16
NC = 2
NS = 16
C = 80
N0 = 94
N1 = 31
E_PAD = NS * (N0 + N1) * C


NB = 4


def _dot_kernel(src_hbm, dst_hbm, ufeat_hbm, ifeat_hbm, out_hbm,
                sidx, didx,
                u0, u1, u2, u3, v0, v1, v2, v3, o0, o1, o2, o3,
                su0, su1, su2, su3, sv0, sv1, sv2, sv3,
                so0, so1, so2, so3):
    cid = lax.axis_index("c")
    sid = lax.axis_index("s")
    row_ids = lax.iota(jnp.int32, L)
    ubuf = (u0, u1, u2, u3)
    vbuf = (v0, v1, v2, v3)
    obuf = (o0, o1, o2, o3)
    usem = (su0, su1, su2, su3)
    vsem = (sv0, sv1, sv2, sv3)
    osem = (so0, so1, so2, so3)

    def run_core(n_chunks, w_base):
        e_per_w = n_chunks * C
        pltpu.sync_copy(src_hbm.at[pl.ds(w_base, e_per_w)],
                        sidx.at[pl.ds(0, e_per_w)])
        pltpu.sync_copy(dst_hbm.at[pl.ds(w_base, e_per_w)],
                        didx.at[pl.ds(0, e_per_w)])

        def fire(i, b):
            off = i * C
            pltpu.async_copy(
                ufeat_hbm.at[sidx.at[pl.ds(off, C)]], ubuf[b], usem[b])
            pltpu.async_copy(
                ifeat_hbm.at[didx.at[pl.ds(off, C)]], vbuf[b], vsem[b])

        def wait_rows(b):
            pltpu.make_async_copy(
                ufeat_hbm.at[sidx.at[pl.ds(0, C)]], ubuf[b], usem[b]).wait()
            pltpu.make_async_copy(
                ifeat_hbm.at[didx.at[pl.ds(0, C)]], vbuf[b], vsem[b]).wait()

        def wait_out(b):
            pltpu.make_async_copy(
                obuf[b], out_hbm.at[pl.ds(0, C)], osem[b]).wait()

        def compute(i, b, guarded):
            wait_rows(b)
            if guarded:
                @pl.when(i + NB - 1 < n_chunks)
                def _():
                    fire(i + NB - 1, (b + NB - 1) % NB)

            @pl.when(i >= NB)
            def _():
                wait_out(b)

            urows = ubuf[b]
            vrows = vbuf[b]

            def group_body(g, _):
                base = g * L
                res = jnp.zeros((L,), jnp.float32)
                for e in range(L):
                    acc = jnp.zeros((L,), jnp.float32)
                    for j in range(NWORD // L):
                        uw = urows[base + e, pl.ds(j * L, L)]
                        vw = vrows[base + e, pl.ds(j * L, L)]
                        uhi = plsc.bitcast(uw, jnp.float32)
                        ulo = plsc.bitcast(uw << 16, jnp.float32)
                        vhi = plsc.bitcast(vw, jnp.float32)
                        vlo = plsc.bitcast(vw << 16, jnp.float32)
                        acc += uhi * vhi
                        acc += ulo * vlo
                    s = jnp.sum(acc)
                    res = jnp.where(row_ids == e, s, res)
                obuf[b][pl.ds(base, L)] = res
                return 0

            lax.fori_loop(0, C // L, group_body, 0)
            pltpu.async_copy(
                obuf[b], out_hbm.at[pl.ds(w_base + i * C, C)], osem[b])

        for j in range(NB - 1):
            fire(j, j)

        def outer_body(o, _):
            for b in range(NB):
                compute(o * NB + b, b, guarded=True)
            return 0

        lax.fori_loop(0, n_chunks // NB, outer_body, 0)
        for r in range(n_chunks % NB):
            i = (n_chunks // NB) * NB + r
            compute(i, i % NB, guarded=False)
        for b in range(NB):
            wait_out(b)

    @pl.when(cid == 0)
    def _():
        run_core(N0, sid * (N0 * C))

    @pl.when(cid == 1)
    def _():
        run_core(N1, NS * (N0 * C) + sid * (N1 * C))


def kernel(ufeat, ifeat, Q, edge_index):
    del Q
    e = edge_index.shape[1]
    assert e <= E_PAD
    src = edge_index[0].astype(jnp.int32)
    dst = edge_index[1].astype(jnp.int32)
    if E_PAD != e:
        src = jnp.pad(src, (0, E_PAD - e))
        dst = jnp.pad(dst, (0, E_PAD - e))

    def pack(x):
        xb = x.astype(jnp.bfloat16)
        hi = lax.bitcast_convert_type(xb[:, :NWORD], jnp.uint16)
        lo = lax.bitcast_convert_type(xb[:, NWORD:], jnp.uint16)
        packed = (hi.astype(jnp.uint32) << 16) | lo.astype(jnp.uint32)
        return lax.bitcast_convert_type(packed, jnp.int32)

    run = pl.kernel(
        _dot_kernel,
        out_type=jax.ShapeDtypeStruct((E_PAD,), jnp.float32),
        mesh=plsc.VectorSubcoreMesh(
            core_axis_name="c", subcore_axis_name="s",
            num_cores=NC, num_subcores=NS),
        scratch_types=[
            pltpu.VMEM((N0 * C,), jnp.int32),
            pltpu.VMEM((N0 * C,), jnp.int32),
            *[pltpu.VMEM((C, NWORD), jnp.int32) for _ in range(2 * NB)],
            *[pltpu.VMEM((C,), jnp.float32) for _ in range(NB)],
            *[pltpu.SemaphoreType.DMA for _ in range(3 * NB)],
        ],
        compiler_params=pltpu.CompilerParams(needs_layout_passes=False),
    )
    out = run(src, dst, pack(ufeat), pack(ifeat))
    return out[:e, None]


# speedup vs baseline: 4.4983x; 1.1885x over previous
import jax
import jax.numpy as jnp
from jax import lax
from jax.experimental import pallas as pl
from jax.experimental.pallas import tpu as pltpu
from jax.experimental.pallas import tpu_sc as plsc

N_FEAT = 256
NWORD = N_FEAT // 2
L = 16
NC = 2
NS = 16
C = 80
N0 = 69
N1 = 56
E_PAD = NS * (N0 + N1) * C


NB = 4


def _dot_kernel(src_hbm, dst_hbm, ufeat_hbm, ifeat_hbm, out_hbm,
                sidx, didx,
                u0, u1, u2, u3, v0, v1, v2, v3, o0, o1, o2, o3,
                su0, su1, su2, su3, sv0, sv1, sv2, sv3,
                so0, so1, so2, so3):
    cid = lax.axis_index("c")
    sid = lax.axis_index("s")
    row_ids = lax.iota(jnp.int32, L)
    ubuf = (u0, u1, u2, u3)
    vbuf = (v0, v1, v2, v3)
    obuf = (o0, o1, o2, o3)
    usem = (su0, su1, su2, su3)
    vsem = (sv0, sv1, sv2, sv3)
    osem = (so0, so1, so2, so3)

    def run_core(n_chunks, w_base):
        e_per_w = n_chunks * C
        pltpu.sync_copy(src_hbm.at[pl.ds(w_base, e_per_w)],
                        sidx.at[pl.ds(0, e_per_w)])
        pltpu.sync_copy(dst_hbm.at[pl.ds(w_base, e_per_w)],
                        didx.at[pl.ds(0, e_per_w)])

        def fire(i, b):
            off = i * C
            pltpu.async_copy(
                ufeat_hbm.at[sidx.at[pl.ds(off, C)]], ubuf[b], usem[b])
            pltpu.async_copy(
                ifeat_hbm.at[didx.at[pl.ds(off, C)]], vbuf[b], vsem[b])

        def wait_rows(b):
            pltpu.make_async_copy(
                ufeat_hbm.at[sidx.at[pl.ds(0, C)]], ubuf[b], usem[b]).wait()
            pltpu.make_async_copy(
                ifeat_hbm.at[didx.at[pl.ds(0, C)]], vbuf[b], vsem[b]).wait()

        def wait_out(b):
            pltpu.make_async_copy(
                obuf[b], out_hbm.at[pl.ds(0, C)], osem[b]).wait()

        def compute(i, b, guarded):
            wait_rows(b)
            if guarded:
                @pl.when(i + NB - 1 < n_chunks)
                def _():
                    fire(i + NB - 1, (b + NB - 1) % NB)

            @pl.when(i >= NB)
            def _():
                wait_out(b)

            urows = ubuf[b]
            vrows = vbuf[b]

            def group_body(g, _):
                base = g * L
                res = jnp.zeros((L,), jnp.float32)
                for e in range(L):
                    acc = jnp.zeros((L,), jnp.float32)
                    for j in range(NWORD // L):
                        uw = urows[base + e, pl.ds(j * L, L)]
                        vw = vrows[base + e, pl.ds(j * L, L)]
                        uhi = plsc.bitcast(uw, jnp.float32)
                        ulo = plsc.bitcast(uw << 16, jnp.float32)
                        vhi = plsc.bitcast(vw, jnp.float32)
                        vlo = plsc.bitcast(vw << 16, jnp.float32)
                        acc += uhi * vhi
                        acc += ulo * vlo
                    s = jnp.sum(acc)
                    res = jnp.where(row_ids == e, s, res)
                obuf[b][pl.ds(base, L)] = res
                return 0

            lax.fori_loop(0, C // L, group_body, 0)
            pltpu.async_copy(
                obuf[b], out_hbm.at[pl.ds(w_base + i * C, C)], osem[b])

        for j in range(NB - 1):
            fire(j, j)

        def outer_body(o, _):
            for b in range(NB):
                compute(o * NB + b, b, guarded=True)
            return 0

        lax.fori_loop(0, n_chunks // NB, outer_body, 0)
        for r in range(n_chunks % NB):
            i = (n_chunks // NB) * NB + r
            compute(i, i % NB, guarded=False)
        for b in range(NB):
            wait_out(b)

    @pl.when(cid == 0)
    def _():
        run_core(N0, sid * (N0 * C))

    @pl.when(cid == 1)
    def _():
        run_core(N1, NS * (N0 * C) + sid * (N1 * C))


def kernel(ufeat, ifeat, Q, edge_index):
    del Q
    e = edge_index.shape[1]
    assert e <= E_PAD
    src = edge_index[0].astype(jnp.int32)
    dst = edge_index[1].astype(jnp.int32)
    if E_PAD != e:
        src = jnp.pad(src, (0, E_PAD - e))
        dst = jnp.pad(dst, (0, E_PAD - e))

    def pack(x):
        xb = x.astype(jnp.bfloat16)
        hi = lax.bitcast_convert_type(xb[:, :NWORD], jnp.uint16)
        lo = lax.bitcast_convert_type(xb[:, NWORD:], jnp.uint16)
        packed = (hi.astype(jnp.uint32) << 16) | lo.astype(jnp.uint32)
        return lax.bitcast_convert_type(packed, jnp.int32)

    run = pl.kernel(
        _dot_kernel,
        out_type=jax.ShapeDtypeStruct((E_PAD,), jnp.float32),
        mesh=plsc.VectorSubcoreMesh(
            core_axis_name="c", subcore_axis_name="s",
            num_cores=NC, num_subcores=NS),
        scratch_types=[
            pltpu.VMEM((N0 * C,), jnp.int32),
            pltpu.VMEM((N0 * C,), jnp.int32),
            *[pltpu.VMEM((C, NWORD), jnp.int32) for _ in range(2 * NB)],
            *[pltpu.VMEM((C,), jnp.float32) for _ in range(NB)],
            *[pltpu.SemaphoreType.DMA for _ in range(3 * NB)],
        ],
        compiler_params=pltpu.CompilerParams(needs_layout_passes=False),
    )
    out = run(src, dst, pack(ufeat), pack(ifeat))
    return out[:e, None]
